# Initial kernel scaffold; baseline (speedup 1.0000x reference)
import jax
import jax.numpy as jnp
from jax.experimental import pallas as pl


def kernel(visit_node_type, table):
    raise NotImplementedError("write your pallas kernel here")


# speedup vs baseline: 4.1543x; 4.1543x over previous
import functools

import jax
import jax.numpy as jnp
from jax import lax
from jax.experimental import pallas as pl
from jax.experimental.pallas import tpu as pltpu
from jax.experimental.pallas import tpu_sc as plsc

EMBED_DIM = 64


def _emb_lookup(idx_flat, table, B):
    info = plsc.get_sparse_core_info()
    NC, NS = info.num_cores, info.num_subcores
    NW = NC * NS
    b_per_w = B // NW
    CH = 1024
    n_ch = b_per_w // CH

    mesh = plsc.VectorSubcoreMesh(core_axis_name="c", subcore_axis_name="s")

    @functools.partial(
        pl.kernel,
        mesh=mesh,
        out_type=jax.ShapeDtypeStruct((B, EMBED_DIM), jnp.float32),
        scratch_types=[
            pltpu.VMEM((CH,), jnp.int32),
            pltpu.VMEM((CH, EMBED_DIM), jnp.float32),
            pltpu.SemaphoreType.DMA,
        ],
        compiler_params=pltpu.CompilerParams(use_tc_tiling_on_sc=False),
    )
    def emb_kernel(idx_hbm, table_hbm, out_hbm, idx_v, rows_v, sem):
        wid = lax.axis_index("s") * NC + lax.axis_index("c")
        base = wid * b_per_w

        def body(i, carry):
            off = base + i * CH
            pltpu.sync_copy(idx_hbm.at[pl.ds(off, CH)], idx_v)
            pltpu.async_copy(table_hbm.at[idx_v], rows_v, sem).wait()
            pltpu.sync_copy(rows_v, out_hbm.at[pl.ds(off, CH)])
            return carry

        lax.fori_loop(0, n_ch, body, 0)

    return emb_kernel(idx_flat, table)


def kernel(visit_node_type, table):
    B0, B1 = visit_node_type.shape
    B = B0 * B1
    idx = visit_node_type.reshape(B).astype(jnp.int32)
    out = _emb_lookup(idx, table, B)
    return out.reshape(B0, B1, EMBED_DIM)


# speedup vs baseline: 4.2899x; 1.0326x over previous
import functools

import jax
import jax.numpy as jnp
from jax import lax
from jax.experimental import pallas as pl
from jax.experimental.pallas import tpu as pltpu
from jax.experimental.pallas import tpu_sc as plsc

EMBED_DIM = 64


def _emb_lookup(idx_flat, table, B):
    info = plsc.get_sparse_core_info()
    NC, NS = info.num_cores, info.num_subcores
    NW = NC * NS
    b_per_w = B // NW
    CH = 800
    nch = b_per_w // CH

    mesh = plsc.VectorSubcoreMesh(core_axis_name="c", subcore_axis_name="s")

    @functools.partial(
        pl.kernel,
        mesh=mesh,
        out_type=jax.ShapeDtypeStruct((B, EMBED_DIM), jnp.float32),
        scratch_types=[
            pltpu.VMEM((CH,), jnp.int32),
            pltpu.VMEM((CH,), jnp.int32),
            pltpu.VMEM((CH, EMBED_DIM), jnp.float32),
            pltpu.VMEM((CH, EMBED_DIM), jnp.float32),
            pltpu.SemaphoreType.DMA,
            pltpu.SemaphoreType.DMA,
            pltpu.SemaphoreType.DMA,
            pltpu.SemaphoreType.DMA,
            pltpu.SemaphoreType.DMA,
            pltpu.SemaphoreType.DMA,
        ],
        compiler_params=pltpu.CompilerParams(use_tc_tiling_on_sc=False),
    )
    def emb_kernel(idx_hbm, table_hbm, out_hbm,
                   idx0, idx1, rows0, rows1,
                   si0, si1, sg0, sg1, ss0, ss1):
        idx_v = (idx0, idx1)
        rows_v = (rows0, rows1)
        si = (si0, si1)
        sg = (sg0, sg1)
        ss = (ss0, ss1)
        wid = lax.axis_index("s") * NC + lax.axis_index("c")
        base = wid * b_per_w

        def start_idx(j, b):
            pltpu.async_copy(idx_hbm.at[pl.ds(base + j * CH, CH)],
                             idx_v[b], si[b])

        def wait_idx(b):
            pltpu.make_async_copy(idx_hbm.at[pl.ds(base, CH)],
                                  idx_v[b], si[b]).wait()

        def start_gather(b):
            pltpu.async_copy(table_hbm.at[idx_v[b]], rows_v[b], sg[b])

        def wait_gather(b):
            pltpu.make_async_copy(table_hbm.at[idx_v[b]],
                                  rows_v[b], sg[b]).wait()

        def start_store(j, b):
            pltpu.async_copy(rows_v[b],
                             out_hbm.at[pl.ds(base + j * CH, CH)], ss[b])

        def wait_store(b):
            pltpu.make_async_copy(rows_v[b],
                                  out_hbm.at[pl.ds(base, CH)], ss[b]).wait()

        def step(j, b):
            o = 1 - b
            wait_idx(b)
            wait_store(b)
            start_gather(b)
            wait_gather(o)
            start_idx(j + 1, o)
            start_store(j - 1, o)

        start_idx(0, 0)
        start_idx(1, 1)
        wait_idx(0)
        start_gather(0)
        wait_idx(1)
        start_gather(1)
        wait_gather(0)
        start_idx(2, 0)
        start_store(0, 0)
        step(2, 0)
        def pair(g, carry):
            j = 3 + 2 * g
            step(j, 1)
            step(j + 1, 0)
            return carry
        lax.fori_loop(0, (nch - 4) // 2, pair, 0)
        wait_idx(1)
        wait_store(1)
        start_gather(1)
        wait_gather(0)
        start_store(nch - 2, 0)
        wait_gather(1)
        start_store(nch - 1, 1)
        wait_store(0)
        wait_store(1)

    return emb_kernel(idx_flat, table)


def kernel(visit_node_type, table):
    B0, B1 = visit_node_type.shape
    B = B0 * B1
    idx = visit_node_type.reshape(B).astype(jnp.int32)
    out = _emb_lookup(idx, table, B)
    return out.reshape(B0, B1, EMBED_DIM)
